# baseline (speedup 1.0000x reference)
import functools
import jax
import jax.numpy as jnp
from jax import lax
from jax.experimental import pallas as pl
from jax.experimental.pallas import tpu as pltpu
from jax.experimental.pallas import tpu_sc as plsc

N = 16384
C = 1000
NC = 2
NS = 16
NW = NC * NS
RPW = N // NW
R = 16
NCH = RPW // R
CW = R * C


def _sc_body(x_hbm, zc_hbm, out_hbm, idx_v, buf_a, buf_b, sem_a, sem_b):
    wid = lax.axis_index("s") * NC + lax.axis_index("c")
    base_row = wid * RPW
    pltpu.sync_copy(x_hbm.at[pl.ds(base_row, RPW)], idx_v)

    lane = lax.iota(jnp.int32, 16)
    row_off = lane * C
    zeros16 = jnp.zeros((16,), jnp.int32)
    ones16 = jnp.ones((16,), jnp.int32)

    pltpu.sync_copy(zc_hbm, buf_a)
    pltpu.sync_copy(zc_hbm, buf_b)

    def flat(c):
        return row_off + idx_v[pl.ds(c * 16, 16)]

    def fill(buf, c):
        plsc.store_scatter(buf, [flat(c)], ones16)

    def reset(buf, c):
        plsc.store_scatter(buf, [flat(c)], zeros16)

    def dma_out(buf, c, sem):
        start = (base_row + c * R) * C
        return pltpu.async_copy(buf, out_hbm.at[pl.ds(start, CW)], sem)

    def wait(buf, sem):
        pltpu.make_async_copy(buf, out_hbm.at[pl.ds(0, CW)], sem).wait()

    fill(buf_a, 0)
    dma_out(buf_a, 0, sem_a)
    fill(buf_b, 1)
    dma_out(buf_b, 1, sem_b)

    def body(i, _):
        c0 = i * 2
        c1 = c0 + 1
        wait(buf_a, sem_a)
        reset(buf_a, c0 - 2)
        fill(buf_a, c0)
        dma_out(buf_a, c0, sem_a)
        wait(buf_b, sem_b)
        reset(buf_b, c1 - 2)
        fill(buf_b, c1)
        dma_out(buf_b, c1, sem_b)
        return 0

    lax.fori_loop(1, NCH // 2, body, 0)

    wait(buf_a, sem_a)
    wait(buf_b, sem_b)


def kernel(x):
    x32 = x.astype(jnp.int32)
    zc = jnp.zeros((CW,), jnp.int32)
    mesh = plsc.VectorSubcoreMesh(core_axis_name="c", subcore_axis_name="s")
    run = functools.partial(
        pl.kernel,
        mesh=mesh,
        compiler_params=pltpu.CompilerParams(
            use_tc_tiling_on_sc=False, needs_layout_passes=False
        ),
        out_type=jax.ShapeDtypeStruct((N * C,), jnp.int32),
        scratch_types=[
            pltpu.VMEM((RPW,), jnp.int32),
            pltpu.VMEM((CW,), jnp.int32),
            pltpu.VMEM((CW,), jnp.int32),
            pltpu.SemaphoreType.DMA,
            pltpu.SemaphoreType.DMA,
        ],
    )(_sc_body)
    return run(x32, zc).reshape(N, C)
